# baseline (speedup 1.0000x reference)
import functools

import jax
import jax.numpy as jnp
import numpy as np
from jax import lax
from jax.experimental import pallas as pl
from jax.experimental.pallas import tpu as pltpu
from jax.experimental.pallas import tpu_sc as plsc

NCORES = 2
NSUB = 16
LANES = 16


def _roberts_sq(x):
    h, w = x.shape
    zrow = jnp.zeros((1, w), jnp.float32)
    zcol = jnp.zeros((h, 1), jnp.float32)
    below = jnp.concatenate([x[1:, :], zrow], axis=0)
    right = jnp.concatenate([x[:, 1:], zcol], axis=1)
    belowright = jnp.concatenate([below[:, 1:], zcol], axis=1)
    gx = x - belowright
    gy = right - below
    return gx * gx + gy * gy + jnp.float32(1e-12)


def _sq_body(x_ref, o_ref):
    o_ref[0] = _roberts_sq(x_ref[0])


def _sq_maps(imgs):
    n, h, w = imgs.shape
    return pl.pallas_call(
        _sq_body,
        grid=(n,),
        in_specs=[pl.BlockSpec((1, h, w), lambda i: (i, 0, 0))],
        out_specs=pl.BlockSpec((1, h, w), lambda i: (i, 0, 0)),
        out_shape=jax.ShapeDtypeStruct((n, h, w), jnp.float32),
    )(imgs)


def _sc_thresholds(sq_flat, *, n_img, hwsz, px, k_rank):
    vecs = px // LANES
    mesh = plsc.VectorSubcoreMesh(
        core_axis_name="c", subcore_axis_name="s",
        num_cores=NCORES, num_subcores=NSUB)

    @functools.partial(
        pl.kernel,
        out_type=jax.ShapeDtypeStruct((NCORES, NSUB, LANES), jnp.int32),
        mesh=mesh,
        compiler_params=pltpu.CompilerParams(needs_layout_passes=False),
        scratch_types=[
            pltpu.VMEM((2 * px,), jnp.float32),
            pltpu.VMEM((16 * 256,), jnp.int32),
            pltpu.VMEM((256,), jnp.int32),
            pltpu.VMEM((NSUB * 256,), jnp.int32),
            pltpu.VMEM((n_img * 32,), jnp.int32),
            pltpu.VMEM((16,), jnp.int32),
            pltpu.VMEM_SHARED((n_img, NSUB * 256), jnp.int32),
            pltpu.VMEM_SHARED((n_img * 32,), jnp.int32),
            pltpu.SemaphoreType.DMA,
            pltpu.SemaphoreType.DMA,
        ],
    )
    def kern(sq_hbm, out_hbm, buf, histo, compact, scanbuf, b1k2, vecst,
             sh_hist, sh_b1k2, sem0, sem1):
        cid = lax.axis_index("c")
        sid = lax.axis_index("s")
        lane256 = lax.iota(jnp.int32, 16) * 256
        ones16 = jnp.ones((16,), jnp.int32)
        kk = jnp.int32(k_rank)

        def shard_off(k):
            return (cid * n_img + k) * hwsz + sid * px

        def zero_histo():
            def z(i, _):
                histo[pl.ds(i * 16, 16)] = jnp.zeros((16,), jnp.int32)
                return 0
            lax.fori_loop(0, 256, z, 0)

        def lane_reduce_and_publish(k):
            def r(vb, _):
                acc = jnp.zeros((16,), jnp.int32)
                for l in range(16):
                    acc = acc + histo[pl.ds(l * 256 + vb * 16, 16)]
                compact[pl.ds(vb * 16, 16)] = acc
                return 0
            lax.fori_loop(0, 16, r, 0)
            pltpu.sync_copy(compact, sh_hist.at[k, pl.ds(sid * 256, 256)])

        def bin1(k, slot):
            zero_histo()
            def b(i, _):
                v = buf[pl.ds(slot * px + i * 16, 16)]
                bits = lax.bitcast_convert_type(v, jnp.int32)
                idx = lane256 + lax.shift_right_logical(bits, 22)
                plsc.addupdate_scatter(histo, [idx], ones16)
                return 0
            lax.fori_loop(0, vecs, b, 0)
            lane_reduce_and_publish(k)

        def bin2(k, slot):
            zero_histo()
            sel = b1k2[pl.ds(k * 32, 16)][0]
            def b(i, _):
                v = buf[pl.ds(slot * px + i * 16, 16)]
                bits = lax.bitcast_convert_type(v, jnp.int32)
                b1v = lax.shift_right_logical(bits, 22)
                bin2v = lax.shift_right_logical(bits, 14) & 0xFF
                idx = lane256 + bin2v
                add = jnp.where(b1v == sel, 1, 0).astype(jnp.int32)
                plsc.addupdate_scatter(histo, [idx], add)
                return 0
            lax.fori_loop(0, vecs, b, 0)
            lane_reduce_and_publish(k)

        def streamed_pass(binfn):
            def outer(j, _):
                k0 = 2 * j
                c0 = pltpu.async_copy(
                    sq_hbm.at[pl.ds(shard_off(k0), px)],
                    buf.at[pl.ds(0, px)], sem0)
                c1 = pltpu.async_copy(
                    sq_hbm.at[pl.ds(shard_off(k0 + 1), px)],
                    buf.at[pl.ds(px, px)], sem1)
                c0.wait()
                binfn(k0, 0)
                c1.wait()
                binfn(k0 + 1, 1)
                return 0
            lax.fori_loop(0, n_img // 2, outer, 0)

        def merged_suffix_stats(rank):
            pltpu.sync_copy(sh_hist.at[sid], scanbuf)
            def m(vb, _):
                acc = jnp.zeros((16,), jnp.int32)
                for t in range(16):
                    acc = acc + scanbuf[pl.ds(t * 256 + vb * 16, 16)]
                compact[pl.ds(vb * 16, 16)] = acc
                return 0
            lax.fori_loop(0, 16, m, 0)

            def s(i, carry):
                tot, cntv, nxtv = carry
                vb = 15 - i
                vec = compact[pl.ds(vb * 16, 16)]
                rc = jnp.cumsum(lax.rev(vec, (0,)))
                suf = lax.rev(rc, (0,)) + tot
                cntv = cntv + jnp.where(suf >= rank, 1, 0).astype(jnp.int32)
                nxtv = jnp.maximum(
                    nxtv, jnp.where(suf < rank, suf, 0).astype(jnp.int32))
                return tot + jnp.sum(vec), cntv, nxtv
            z16 = jnp.zeros((16,), jnp.int32)
            _, cntv, nxtv = lax.fori_loop(0, 16, s, (jnp.int32(0), z16, z16))
            return jnp.sum(cntv) - 1, jnp.max(nxtv)

        streamed_pass(bin1)
        plsc.subcore_barrier()

        @pl.when(sid < n_img)
        def _scan1():
            b1, nxt = merged_suffix_stats(kk)
            k2 = kk - nxt
            vecst[...] = jnp.full((16,), b1, jnp.int32)
            pltpu.sync_copy(vecst, sh_b1k2.at[pl.ds(sid * 32, 16)])
            vecst[...] = jnp.full((16,), k2, jnp.int32)
            pltpu.sync_copy(vecst, sh_b1k2.at[pl.ds(sid * 32 + 16, 16)])

        plsc.subcore_barrier()
        pltpu.sync_copy(sh_b1k2, b1k2)

        streamed_pass(bin2)
        plsc.subcore_barrier()

        vecst[...] = jnp.zeros((16,), jnp.int32)

        @pl.when(sid < n_img)
        def _scan2():
            k2 = b1k2[pl.ds(sid * 32 + 16, 16)][0]
            b2, _ = merged_suffix_stats(k2)
            b1 = b1k2[pl.ds(sid * 32, 16)][0]
            thr = lax.shift_left(b1, 22) | lax.shift_left(b2, 14)
            vecst[...] = jnp.full((16,), thr, jnp.int32)

        pltpu.sync_copy(vecst, out_hbm.at[cid, sid])

    return kern(sq_flat)


def _merge_body(tsq_ref, psq_ref, thr_ref, out_ref, tf_ref, pf_ref, acc_ref):
    i = pl.program_id(0)
    n = pl.num_programs(0)

    @pl.when(i == 0)
    def _init():
        tf_ref[...] = jnp.zeros_like(tf_ref)
        pf_ref[...] = jnp.zeros_like(pf_ref)
        acc_ref[0] = jnp.float32(0.0)

    tsq = tsq_ref[0]
    psq = psq_ref[0]
    tbits = lax.bitcast_convert_type(tsq, jnp.int32)
    pbits = lax.bitcast_convert_type(psq, jnp.int32)
    tf = jnp.where(tbits >= thr_ref[i], jnp.sqrt(tsq), tf_ref[...])
    pf = jnp.where(pbits >= thr_ref[n + i], jnp.sqrt(psq), pf_ref[...])
    tf_ref[...] = tf
    pf_ref[...] = pf
    e = jnp.abs((tf - pf) / (tf + pf + jnp.float32(1e-5)))
    acc_ref[0] += jnp.sum(e)

    @pl.when(i == n - 1)
    def _fin():
        out_ref[0] = acc_ref[0]


def kernel(predictions, target, alpha):
    b, c, h, w = predictions.shape
    n = b * c
    hw = h * w
    ax = int(np.floor(0.1 * hw))
    px = (h // NSUB) * w

    imgs = jnp.concatenate(
        [target.reshape(n, h, w), predictions.reshape(n, h, w)], axis=0)
    sq = _sq_maps(imgs)
    thr_raw = _sc_thresholds(
        sq.reshape(-1), n_img=n, hwsz=hw, px=px, k_rank=ax)
    thrs = thr_raw[:, :n, 0].reshape(2 * n)

    total = pl.pallas_call(
        _merge_body,
        grid=(n,),
        in_specs=[
            pl.BlockSpec((1, h, w), lambda i: (i, 0, 0)),
            pl.BlockSpec((1, h, w), lambda i: (i + n, 0, 0)),
            pl.BlockSpec(memory_space=pltpu.SMEM),
        ],
        out_specs=pl.BlockSpec(memory_space=pltpu.SMEM),
        out_shape=jax.ShapeDtypeStruct((1,), jnp.float32),
        scratch_shapes=[
            pltpu.VMEM((h, w), jnp.float32),
            pltpu.VMEM((h, w), jnp.float32),
            pltpu.SMEM((1,), jnp.float32),
        ],
    )(sq, sq, thrs)
    return alpha * total[0] / jnp.float32(n * hw)


# speedup vs baseline: 1.0813x; 1.0813x over previous
import functools

import jax
import jax.numpy as jnp
import numpy as np
from jax import lax
from jax.experimental import pallas as pl
from jax.experimental.pallas import tpu as pltpu
from jax.experimental.pallas import tpu_sc as plsc

NCORES = 2
NSUB = 16
LANES = 16


def _roberts_sq(x):
    h, w = x.shape
    zrow = jnp.zeros((1, w), jnp.float32)
    zcol = jnp.zeros((h, 1), jnp.float32)
    below = jnp.concatenate([x[1:, :], zrow], axis=0)
    right = jnp.concatenate([x[:, 1:], zcol], axis=1)
    belowright = jnp.concatenate([below[:, 1:], zcol], axis=1)
    gx = x - belowright
    gy = right - below
    return gx * gx + gy * gy + jnp.float32(1e-12)


def _sq_body(x_ref, o_ref):
    o_ref[0] = _roberts_sq(x_ref[0])


def _sq_maps(imgs):
    n, h, w = imgs.shape
    return pl.pallas_call(
        _sq_body,
        grid=(n,),
        in_specs=[pl.BlockSpec((1, h, w), lambda i: (i, 0, 0))],
        out_specs=pl.BlockSpec((1, h, w), lambda i: (i, 0, 0)),
        out_shape=jax.ShapeDtypeStruct((n, h, w), jnp.float32),
    )(imgs)


def _sc_thresholds(sq_flat, *, n_img, hwsz, px, k_rank):
    vecs = px // LANES
    mesh = plsc.VectorSubcoreMesh(
        core_axis_name="c", subcore_axis_name="s",
        num_cores=NCORES, num_subcores=NSUB)

    @functools.partial(
        pl.kernel,
        out_type=jax.ShapeDtypeStruct((NCORES, NSUB, LANES), jnp.int32),
        mesh=mesh,
        compiler_params=pltpu.CompilerParams(needs_layout_passes=False),
        scratch_types=[
            pltpu.VMEM((2 * px,), jnp.float32),
            pltpu.VMEM((16 * 256,), jnp.int32),
            pltpu.VMEM((256,), jnp.int32),
            pltpu.VMEM((NSUB * 256,), jnp.int32),
            pltpu.VMEM((n_img * 32,), jnp.int32),
            pltpu.VMEM((16,), jnp.int32),
            pltpu.VMEM_SHARED((n_img, NSUB * 256), jnp.int32),
            pltpu.VMEM_SHARED((n_img * 32,), jnp.int32),
            pltpu.SemaphoreType.DMA,
            pltpu.SemaphoreType.DMA,
        ],
    )
    def kern(sq_hbm, out_hbm, buf, histo, compact, scanbuf, b1k2, vecst,
             sh_hist, sh_b1k2, sem0, sem1):
        cid = lax.axis_index("c")
        sid = lax.axis_index("s")
        lane256 = lax.iota(jnp.int32, 16) * 256
        ones16 = jnp.ones((16,), jnp.int32)
        kk = jnp.int32(k_rank)

        def shard_off(k):
            return (cid * n_img + k) * hwsz + sid * px

        def zero_histo():
            def z(i, _):
                histo[pl.ds(i * 16, 16)] = jnp.zeros((16,), jnp.int32)
                return 0
            lax.fori_loop(0, 256, z, 0, unroll=16)

        def lane_reduce_and_publish(k):
            def r(vb, _):
                acc = jnp.zeros((16,), jnp.int32)
                for l in range(16):
                    acc = acc + histo[pl.ds(l * 256 + vb * 16, 16)]
                compact[pl.ds(vb * 16, 16)] = acc
                return 0
            lax.fori_loop(0, 16, r, 0, unroll=4)
            pltpu.sync_copy(compact, sh_hist.at[k, pl.ds(sid * 256, 256)])

        def bin1(k, slot):
            zero_histo()
            def b(i, _):
                v = buf[pl.ds(slot * px + i * 16, 16)]
                bits = lax.bitcast_convert_type(v, jnp.int32)
                idx = lane256 + lax.shift_right_logical(bits, 22)
                plsc.addupdate_scatter(histo, [idx], ones16)
                return 0
            lax.fori_loop(0, vecs, b, 0, unroll=8)
            lane_reduce_and_publish(k)

        def bin2(k, slot):
            zero_histo()
            sel = b1k2[pl.ds(k * 32, 16)][0]
            def b(i, _):
                v = buf[pl.ds(slot * px + i * 16, 16)]
                bits = lax.bitcast_convert_type(v, jnp.int32)
                b1v = lax.shift_right_logical(bits, 22)
                bin2v = lax.shift_right_logical(bits, 14) & 0xFF
                idx = lane256 + bin2v
                add = jnp.where(b1v == sel, 1, 0).astype(jnp.int32)
                plsc.addupdate_scatter(histo, [idx], add)
                return 0
            lax.fori_loop(0, vecs, b, 0, unroll=8)
            lane_reduce_and_publish(k)

        def streamed_pass(binfn):
            def outer(j, _):
                k0 = 2 * j
                c0 = pltpu.async_copy(
                    sq_hbm.at[pl.ds(shard_off(k0), px)],
                    buf.at[pl.ds(0, px)], sem0)
                c1 = pltpu.async_copy(
                    sq_hbm.at[pl.ds(shard_off(k0 + 1), px)],
                    buf.at[pl.ds(px, px)], sem1)
                c0.wait()
                binfn(k0, 0)
                c1.wait()
                binfn(k0 + 1, 1)
                return 0
            lax.fori_loop(0, n_img // 2, outer, 0)

        def merged_suffix_stats(rank):
            pltpu.sync_copy(sh_hist.at[sid], scanbuf)
            def m(vb, _):
                acc = jnp.zeros((16,), jnp.int32)
                for t in range(16):
                    acc = acc + scanbuf[pl.ds(t * 256 + vb * 16, 16)]
                compact[pl.ds(vb * 16, 16)] = acc
                return 0
            lax.fori_loop(0, 16, m, 0, unroll=4)

            def s(i, carry):
                tot, cntv, nxtv = carry
                vb = 15 - i
                vec = compact[pl.ds(vb * 16, 16)]
                rc = jnp.cumsum(lax.rev(vec, (0,)))
                suf = lax.rev(rc, (0,)) + tot
                cntv = cntv + jnp.where(suf >= rank, 1, 0).astype(jnp.int32)
                nxtv = jnp.maximum(
                    nxtv, jnp.where(suf < rank, suf, 0).astype(jnp.int32))
                return tot + jnp.sum(vec), cntv, nxtv
            z16 = jnp.zeros((16,), jnp.int32)
            _, cntv, nxtv = lax.fori_loop(0, 16, s, (jnp.int32(0), z16, z16), unroll=4)
            return jnp.sum(cntv) - 1, jnp.max(nxtv)

        streamed_pass(bin1)
        plsc.subcore_barrier()

        @pl.when(sid < n_img)
        def _scan1():
            b1, nxt = merged_suffix_stats(kk)
            k2 = kk - nxt
            vecst[...] = jnp.full((16,), b1, jnp.int32)
            pltpu.sync_copy(vecst, sh_b1k2.at[pl.ds(sid * 32, 16)])
            vecst[...] = jnp.full((16,), k2, jnp.int32)
            pltpu.sync_copy(vecst, sh_b1k2.at[pl.ds(sid * 32 + 16, 16)])

        plsc.subcore_barrier()
        pltpu.sync_copy(sh_b1k2, b1k2)

        streamed_pass(bin2)
        plsc.subcore_barrier()

        vecst[...] = jnp.zeros((16,), jnp.int32)

        @pl.when(sid < n_img)
        def _scan2():
            k2 = b1k2[pl.ds(sid * 32 + 16, 16)][0]
            b2, _ = merged_suffix_stats(k2)
            b1 = b1k2[pl.ds(sid * 32, 16)][0]
            thr = lax.shift_left(b1, 22) | lax.shift_left(b2, 14)
            vecst[...] = jnp.full((16,), thr, jnp.int32)

        pltpu.sync_copy(vecst, out_hbm.at[cid, sid])

    return kern(sq_flat)


def _merge_body(tsq_ref, psq_ref, thr_ref, out_ref, tf_ref, pf_ref, acc_ref):
    i = pl.program_id(0)
    n = pl.num_programs(0)

    @pl.when(i == 0)
    def _init():
        tf_ref[...] = jnp.zeros_like(tf_ref)
        pf_ref[...] = jnp.zeros_like(pf_ref)
        acc_ref[0] = jnp.float32(0.0)

    tsq = tsq_ref[0]
    psq = psq_ref[0]
    tbits = lax.bitcast_convert_type(tsq, jnp.int32)
    pbits = lax.bitcast_convert_type(psq, jnp.int32)
    tf = jnp.where(tbits >= thr_ref[i], jnp.sqrt(tsq), tf_ref[...])
    pf = jnp.where(pbits >= thr_ref[n + i], jnp.sqrt(psq), pf_ref[...])
    tf_ref[...] = tf
    pf_ref[...] = pf
    e = jnp.abs((tf - pf) / (tf + pf + jnp.float32(1e-5)))
    acc_ref[0] += jnp.sum(e)

    @pl.when(i == n - 1)
    def _fin():
        out_ref[0] = acc_ref[0]


def kernel(predictions, target, alpha):
    b, c, h, w = predictions.shape
    n = b * c
    hw = h * w
    ax = int(np.floor(0.1 * hw))
    px = (h // NSUB) * w

    imgs = jnp.concatenate(
        [target.reshape(n, h, w), predictions.reshape(n, h, w)], axis=0)
    sq = _sq_maps(imgs)
    thr_raw = _sc_thresholds(
        sq.reshape(-1), n_img=n, hwsz=hw, px=px, k_rank=ax)
    thrs = thr_raw[:, :n, 0].reshape(2 * n)

    total = pl.pallas_call(
        _merge_body,
        grid=(n,),
        in_specs=[
            pl.BlockSpec((1, h, w), lambda i: (i, 0, 0)),
            pl.BlockSpec((1, h, w), lambda i: (i + n, 0, 0)),
            pl.BlockSpec(memory_space=pltpu.SMEM),
        ],
        out_specs=pl.BlockSpec(memory_space=pltpu.SMEM),
        out_shape=jax.ShapeDtypeStruct((1,), jnp.float32),
        scratch_shapes=[
            pltpu.VMEM((h, w), jnp.float32),
            pltpu.VMEM((h, w), jnp.float32),
            pltpu.SMEM((1,), jnp.float32),
        ],
    )(sq, sq, thrs)
    return alpha * total[0] / jnp.float32(n * hw)


# speedup vs baseline: 1.0856x; 1.0040x over previous
import functools

import jax
import jax.numpy as jnp
import numpy as np
from jax import lax
from jax.experimental import pallas as pl
from jax.experimental.pallas import tpu as pltpu
from jax.experimental.pallas import tpu_sc as plsc

NCORES = 2
NSUB = 16
LANES = 16


def _roberts_sq(x):
    h, w = x.shape
    zrow = jnp.zeros((1, w), jnp.float32)
    zcol = jnp.zeros((h, 1), jnp.float32)
    below = jnp.concatenate([x[1:, :], zrow], axis=0)
    right = jnp.concatenate([x[:, 1:], zcol], axis=1)
    belowright = jnp.concatenate([below[:, 1:], zcol], axis=1)
    gx = x - belowright
    gy = right - below
    return gx * gx + gy * gy + jnp.float32(1e-12)


def _sq_body(x_ref, o_ref):
    o_ref[0] = _roberts_sq(x_ref[0])


def _sq_maps(imgs):
    n, h, w = imgs.shape
    return pl.pallas_call(
        _sq_body,
        grid=(n,),
        in_specs=[pl.BlockSpec((1, h, w), lambda i: (i, 0, 0))],
        out_specs=pl.BlockSpec((1, h, w), lambda i: (i, 0, 0)),
        out_shape=jax.ShapeDtypeStruct((n, h, w), jnp.float32),
    )(imgs)


def _sc_thresholds(sq_flat, *, n_img, hwsz, px, k_rank):
    vecs = px // LANES
    mesh = plsc.VectorSubcoreMesh(
        core_axis_name="c", subcore_axis_name="s",
        num_cores=NCORES, num_subcores=NSUB)

    @functools.partial(
        pl.kernel,
        out_type=jax.ShapeDtypeStruct((NCORES, NSUB, LANES), jnp.int32),
        mesh=mesh,
        compiler_params=pltpu.CompilerParams(needs_layout_passes=False),
        scratch_types=[
            pltpu.VMEM((2 * px,), jnp.float32),
            pltpu.VMEM((16 * 256,), jnp.int32),
            pltpu.VMEM((256,), jnp.int32),
            pltpu.VMEM((NSUB * 256,), jnp.int32),
            pltpu.VMEM((n_img * 32,), jnp.int32),
            pltpu.VMEM((16,), jnp.int32),
            pltpu.VMEM_SHARED((n_img, NSUB * 256), jnp.int32),
            pltpu.VMEM_SHARED((n_img * 32,), jnp.int32),
            pltpu.SemaphoreType.DMA,
            pltpu.SemaphoreType.DMA,
        ],
    )
    def kern(sq_hbm, out_hbm, buf, histo, compact, scanbuf, b1k2, vecst,
             sh_hist, sh_b1k2, sem0, sem1):
        cid = lax.axis_index("c")
        sid = lax.axis_index("s")
        lane256 = lax.iota(jnp.int32, 16) * 256
        ones16 = jnp.ones((16,), jnp.int32)
        kk = jnp.int32(k_rank)

        def shard_off(k):
            return (cid * n_img + k) * hwsz + sid * px

        def zero_histo():
            def z(i, _):
                histo[pl.ds(i * 16, 16)] = jnp.zeros((16,), jnp.int32)
                return 0
            lax.fori_loop(0, 256, z, 0, unroll=16)

        def lane_reduce_and_publish(k):
            def r(vb, _):
                acc = jnp.zeros((16,), jnp.int32)
                for l in range(16):
                    acc = acc + histo[pl.ds(l * 256 + vb * 16, 16)]
                compact[pl.ds(vb * 16, 16)] = acc
                return 0
            lax.fori_loop(0, 16, r, 0)
            pltpu.sync_copy(compact, sh_hist.at[k, pl.ds(sid * 256, 256)])

        def bin1(k, slot):
            zero_histo()
            def b(i, _):
                v = buf[pl.ds(slot * px + i * 16, 16)]
                bits = lax.bitcast_convert_type(v, jnp.int32)
                idx = lane256 + lax.shift_right_logical(bits, 22)
                plsc.addupdate_scatter(histo, [idx], ones16)
                return 0
            lax.fori_loop(0, vecs, b, 0, unroll=8)
            lane_reduce_and_publish(k)

        def bin2(k, slot):
            zero_histo()
            sel = b1k2[pl.ds(k * 32, 16)][0]
            def b(i, _):
                v = buf[pl.ds(slot * px + i * 16, 16)]
                bits = lax.bitcast_convert_type(v, jnp.int32)
                b1v = lax.shift_right_logical(bits, 22)
                bin2v = lax.shift_right_logical(bits, 14) & 0xFF
                idx = lane256 + bin2v
                add = jnp.where(b1v == sel, 1, 0).astype(jnp.int32)
                plsc.addupdate_scatter(histo, [idx], add)
                return 0
            lax.fori_loop(0, vecs, b, 0, unroll=8)
            lane_reduce_and_publish(k)

        def streamed_pass(binfn):
            def outer(j, _):
                k0 = 2 * j
                c0 = pltpu.async_copy(
                    sq_hbm.at[pl.ds(shard_off(k0), px)],
                    buf.at[pl.ds(0, px)], sem0)
                c1 = pltpu.async_copy(
                    sq_hbm.at[pl.ds(shard_off(k0 + 1), px)],
                    buf.at[pl.ds(px, px)], sem1)
                c0.wait()
                binfn(k0, 0)
                c1.wait()
                binfn(k0 + 1, 1)
                return 0
            lax.fori_loop(0, n_img // 2, outer, 0)

        def merged_suffix_stats(rank):
            pltpu.sync_copy(sh_hist.at[sid], scanbuf)
            def m(vb, _):
                acc = jnp.zeros((16,), jnp.int32)
                for t in range(16):
                    acc = acc + scanbuf[pl.ds(t * 256 + vb * 16, 16)]
                compact[pl.ds(vb * 16, 16)] = acc
                return 0
            lax.fori_loop(0, 16, m, 0)

            def s(i, carry):
                tot, cntv, nxtv = carry
                vb = 15 - i
                vec = compact[pl.ds(vb * 16, 16)]
                rc = jnp.cumsum(lax.rev(vec, (0,)))
                suf = lax.rev(rc, (0,)) + tot
                cntv = cntv + jnp.where(suf >= rank, 1, 0).astype(jnp.int32)
                nxtv = jnp.maximum(
                    nxtv, jnp.where(suf < rank, suf, 0).astype(jnp.int32))
                return tot + jnp.sum(vec), cntv, nxtv
            z16 = jnp.zeros((16,), jnp.int32)
            _, cntv, nxtv = lax.fori_loop(0, 16, s, (jnp.int32(0), z16, z16))
            return jnp.sum(cntv) - 1, jnp.max(nxtv)

        streamed_pass(bin1)
        plsc.subcore_barrier()

        @pl.when(sid < n_img)
        def _scan1():
            b1, nxt = merged_suffix_stats(kk)
            k2 = kk - nxt
            vecst[...] = jnp.full((16,), b1, jnp.int32)
            pltpu.sync_copy(vecst, sh_b1k2.at[pl.ds(sid * 32, 16)])
            vecst[...] = jnp.full((16,), k2, jnp.int32)
            pltpu.sync_copy(vecst, sh_b1k2.at[pl.ds(sid * 32 + 16, 16)])

        plsc.subcore_barrier()
        pltpu.sync_copy(sh_b1k2, b1k2)

        streamed_pass(bin2)
        plsc.subcore_barrier()

        vecst[...] = jnp.zeros((16,), jnp.int32)

        @pl.when(sid < n_img)
        def _scan2():
            k2 = b1k2[pl.ds(sid * 32 + 16, 16)][0]
            b2, _ = merged_suffix_stats(k2)
            b1 = b1k2[pl.ds(sid * 32, 16)][0]
            thr = lax.shift_left(b1, 22) | lax.shift_left(b2, 14)
            vecst[...] = jnp.full((16,), thr, jnp.int32)

        pltpu.sync_copy(vecst, out_hbm.at[cid, sid])

    return kern(sq_flat)


def _merge_body(tsq_ref, psq_ref, thr_ref, out_ref, tf_ref, pf_ref, acc_ref):
    i = pl.program_id(0)
    n = pl.num_programs(0)

    @pl.when(i == 0)
    def _init():
        tf_ref[...] = jnp.zeros_like(tf_ref)
        pf_ref[...] = jnp.zeros_like(pf_ref)
        acc_ref[0] = jnp.float32(0.0)

    tsq = tsq_ref[0]
    psq = psq_ref[0]
    tbits = lax.bitcast_convert_type(tsq, jnp.int32)
    pbits = lax.bitcast_convert_type(psq, jnp.int32)
    tf = jnp.where(tbits >= thr_ref[i], jnp.sqrt(tsq), tf_ref[...])
    pf = jnp.where(pbits >= thr_ref[n + i], jnp.sqrt(psq), pf_ref[...])
    tf_ref[...] = tf
    pf_ref[...] = pf
    e = jnp.abs((tf - pf) / (tf + pf + jnp.float32(1e-5)))
    acc_ref[0] += jnp.sum(e)

    @pl.when(i == n - 1)
    def _fin():
        out_ref[0] = acc_ref[0]


def kernel(predictions, target, alpha):
    b, c, h, w = predictions.shape
    n = b * c
    hw = h * w
    ax = int(np.floor(0.1 * hw))
    px = (h // NSUB) * w

    imgs = jnp.concatenate(
        [target.reshape(n, h, w), predictions.reshape(n, h, w)], axis=0)
    sq = _sq_maps(imgs)
    thr_raw = _sc_thresholds(
        sq.reshape(-1), n_img=n, hwsz=hw, px=px, k_rank=ax)
    thrs = thr_raw[:, :n, 0].reshape(2 * n)

    total = pl.pallas_call(
        _merge_body,
        grid=(n,),
        in_specs=[
            pl.BlockSpec((1, h, w), lambda i: (i, 0, 0)),
            pl.BlockSpec((1, h, w), lambda i: (i + n, 0, 0)),
            pl.BlockSpec(memory_space=pltpu.SMEM),
        ],
        out_specs=pl.BlockSpec(memory_space=pltpu.SMEM),
        out_shape=jax.ShapeDtypeStruct((1,), jnp.float32),
        scratch_shapes=[
            pltpu.VMEM((h, w), jnp.float32),
            pltpu.VMEM((h, w), jnp.float32),
            pltpu.SMEM((1,), jnp.float32),
        ],
    )(sq, sq, thrs)
    return alpha * total[0] / jnp.float32(n * hw)


# speedup vs baseline: 2.3281x; 2.1445x over previous
import functools

import jax
import jax.numpy as jnp
import numpy as np
from jax import lax
from jax.experimental import pallas as pl
from jax.experimental.pallas import tpu as pltpu
from jax.experimental.pallas import tpu_sc as plsc

NCORES = 2
NSUB = 16
LANES = 16


def _roberts_sq(x):
    h, w = x.shape
    zrow = jnp.zeros((1, w), jnp.float32)
    zcol = jnp.zeros((h, 1), jnp.float32)
    below = jnp.concatenate([x[1:, :], zrow], axis=0)
    right = jnp.concatenate([x[:, 1:], zcol], axis=1)
    belowright = jnp.concatenate([below[:, 1:], zcol], axis=1)
    gx = x - belowright
    gy = right - below
    return gx * gx + gy * gy + jnp.float32(1e-12)


def _sq_body(x_ref, o_ref, ob_ref):
    sq = _roberts_sq(x_ref[0])
    o_ref[0] = sq
    ob_ref[0] = sq.astype(jnp.bfloat16)


def _sq_maps(imgs):
    n, h, w = imgs.shape
    return pl.pallas_call(
        _sq_body,
        grid=(n,),
        in_specs=[pl.BlockSpec((1, h, w), lambda i: (i, 0, 0))],
        out_specs=[pl.BlockSpec((1, h, w), lambda i: (i, 0, 0)),
                   pl.BlockSpec((1, h, w), lambda i: (i, 0, 0))],
        out_shape=[jax.ShapeDtypeStruct((n, h, w), jnp.float32),
                   jax.ShapeDtypeStruct((n, h, w), jnp.bfloat16)],
    )(imgs)


def _sc_thresholds(sq, *, n_img, hwsz, px, k_rank):
    vecs = px // 32
    mesh = plsc.VectorSubcoreMesh(
        core_axis_name="c", subcore_axis_name="s",
        num_cores=NCORES, num_subcores=NSUB)

    @functools.partial(
        pl.kernel,
        out_type=jax.ShapeDtypeStruct((NCORES, NSUB, LANES), jnp.int32),
        mesh=mesh,
        compiler_params=pltpu.CompilerParams(needs_layout_passes=False),
        scratch_types=[
            pltpu.VMEM((4, px // 512, 512), jnp.bfloat16),
            [pltpu.VMEM((16 * 256,), jnp.int32)] * 2,
            pltpu.VMEM((256,), jnp.int32),
            pltpu.VMEM((NSUB * 256,), jnp.int32),
            pltpu.VMEM((n_img * 32,), jnp.int32),
            pltpu.VMEM((16,), jnp.int32),
            pltpu.VMEM_SHARED((n_img, NSUB * 256), jnp.int32),
            pltpu.VMEM_SHARED((n_img * 32,), jnp.int32),
            [pltpu.SemaphoreType.DMA] * 4,
        ],
    )
    def kern(sq_hbm, out_hbm, buf, hists, compact, scanbuf, b1k2, vecst,
             sh_hist, sh_b1k2, sems):
        cid = lax.axis_index("c")
        sid = lax.axis_index("s")
        lane128 = lax.iota(jnp.int32, 16) * 128
        lane256 = lax.iota(jnp.int32, 16) * 256
        ones16 = jnp.ones((16,), jnp.int32)
        kk = jnp.int32(k_rank)

        rows = px // 512

        def zero_histo(nb):
            @plsc.parallel_loop(0, nb, unroll=4)
            def _z(i):
                for hh in hists:
                    hh[pl.ds(i * 16, 16)] = jnp.zeros((16,), jnp.int32)

        def lane_reduce_and_publish(k, nb):
            @plsc.parallel_loop(0, nb // 16)
            def _r(vb):
                accs = [jnp.zeros((16,), jnp.int32) for _ in hists]
                for l in range(16):
                    for hj, hh in enumerate(hists):
                        accs[hj] = accs[hj] + hh[pl.ds(l * nb + vb * 16, 16)]
                compact[pl.ds(vb * 16, 16)] = accs[0] + accs[1]
            pltpu.sync_copy(compact.at[pl.ds(0, nb)],
                            sh_hist.at[k, pl.ds(sid * 256, nb)])

        def bin1(k, slot):
            zero_histo(256)
            @plsc.parallel_loop(0, vecs // 2, unroll=4)
            def _b1(iv):
                for j, hh in enumerate(hists):
                    i = iv * 2 + j
                    r = lax.shift_right_logical(i, 4)
                    cc = (i & 15) * 32
                    v = buf[slot, r, pl.ds(cc, 32)]
                    bits16 = lax.bitcast_convert_type(v, jnp.int16)
                    pa, pb = plsc.unpack(
                        bits16, format=plsc.PackFormat.INTERLEAVED,
                        preferred_element_type=jnp.int32)
                    ia = lane256 + lax.shift_right_logical(pa, 7)
                    ib = lane256 + lax.shift_right_logical(pb, 7)
                    plsc.addupdate_scatter(hh, [ia], ones16)
                    plsc.addupdate_scatter(hists[1 - j], [ib], ones16)
            lane_reduce_and_publish(k, 256)

        def bin2(k, slot):
            zero_histo(128)
            sel = b1k2[pl.ds(k * 32, 16)][0]
            @plsc.parallel_loop(0, vecs // 2, unroll=4)
            def _b2(iv):
                for j, hh in enumerate(hists):
                    i = iv * 2 + j
                    r = lax.shift_right_logical(i, 4)
                    cc = (i & 15) * 32
                    v = buf[slot, r, pl.ds(cc, 32)]
                    bits16 = lax.bitcast_convert_type(v, jnp.int16)
                    pa, pb = plsc.unpack(
                        bits16, format=plsc.PackFormat.INTERLEAVED,
                        preferred_element_type=jnp.int32)
                    for hx, half in ((hh, pa), (hists[1 - j], pb)):
                        b1v = lax.shift_right_logical(half, 7)
                        idx = lane128 + (half & 0x7F)
                        add = jnp.where(b1v == sel, 1, 0).astype(jnp.int32)
                        plsc.addupdate_scatter(hx, [idx], add)
            lane_reduce_and_publish(k, 128)

        def streamed_pass(binfn):
            for t in range(4):
                pltpu.async_copy(
                    sq_hbm.at[cid * n_img + t, pl.ds(sid * rows, rows)],
                    buf.at[t], sems[t])

            def outer(q, _):
                for t in range(4):
                    k = q * 4 + t
                    pltpu.make_async_copy(
                        sq_hbm.at[0, pl.ds(sid * rows, rows)],
                        buf.at[t], sems[t]).wait()
                    binfn(k, t)

                    @pl.when(k + 4 < n_img)
                    def _prefetch():
                        pltpu.async_copy(
                            sq_hbm.at[cid * n_img + k + 4,
                                      pl.ds(sid * rows, rows)],
                            buf.at[t], sems[t])
                return 0
            lax.fori_loop(0, n_img // 4, outer, 0)

        def merged_suffix_stats(rank, nb):
            pltpu.sync_copy(sh_hist.at[sid], scanbuf)
            def m(vb, _):
                acc = jnp.zeros((16,), jnp.int32)
                for t in range(16):
                    acc = acc + scanbuf[pl.ds(t * 256 + vb * 16, 16)]
                compact[pl.ds(vb * 16, 16)] = acc
                return 0
            lax.fori_loop(0, nb // 16, m, 0)

            def s(i, carry):
                tot, cntv, nxtv = carry
                vb = nb // 16 - 1 - i
                vec = compact[pl.ds(vb * 16, 16)]
                rc = jnp.cumsum(lax.rev(vec, (0,)))
                suf = lax.rev(rc, (0,)) + tot
                cntv = cntv + jnp.where(suf >= rank, 1, 0).astype(jnp.int32)
                nxtv = jnp.maximum(
                    nxtv, jnp.where(suf < rank, suf, 0).astype(jnp.int32))
                return tot + jnp.sum(vec), cntv, nxtv
            z16 = jnp.zeros((16,), jnp.int32)
            _, cntv, nxtv = lax.fori_loop(
                0, nb // 16, s, (jnp.int32(0), z16, z16))
            return jnp.sum(cntv) - 1, jnp.max(nxtv)

        streamed_pass(bin1)
        plsc.subcore_barrier()

        @pl.when(sid < n_img)
        def _scan1():
            b1, nxt = merged_suffix_stats(kk, 256)
            k2 = kk - nxt
            vecst[...] = jnp.full((16,), b1, jnp.int32)
            pltpu.sync_copy(vecst, sh_b1k2.at[pl.ds(sid * 32, 16)])
            vecst[...] = jnp.full((16,), k2, jnp.int32)
            pltpu.sync_copy(vecst, sh_b1k2.at[pl.ds(sid * 32 + 16, 16)])

        plsc.subcore_barrier()
        pltpu.sync_copy(sh_b1k2, b1k2)

        streamed_pass(bin2)
        plsc.subcore_barrier()

        vecst[...] = jnp.zeros((16,), jnp.int32)

        @pl.when(sid < n_img)
        def _scan2():
            k2 = b1k2[pl.ds(sid * 32 + 16, 16)][0]
            b2, _ = merged_suffix_stats(k2, 128)
            b1 = b1k2[pl.ds(sid * 32, 16)][0]
            thr = lax.shift_left(lax.shift_left(b1, 7) | b2, 16)
            vecst[...] = jnp.full((16,), thr, jnp.int32)

        pltpu.sync_copy(vecst, out_hbm.at[cid, sid])

    return kern(sq)


def _merge_body(tsq_ref, psq_ref, thr_ref, out_ref, tf_ref, pf_ref, acc_ref):
    i = pl.program_id(0)
    n = pl.num_programs(0)

    @pl.when(i == 0)
    def _init():
        tf_ref[...] = jnp.zeros_like(tf_ref)
        pf_ref[...] = jnp.zeros_like(pf_ref)
        acc_ref[0] = jnp.float32(0.0)

    tsq = tsq_ref[0]
    psq = psq_ref[0]
    tbits = lax.bitcast_convert_type(tsq, jnp.int32)
    pbits = lax.bitcast_convert_type(psq, jnp.int32)
    tf = jnp.where(tbits >= thr_ref[i], jnp.sqrt(tsq), tf_ref[...])
    pf = jnp.where(pbits >= thr_ref[n + i], jnp.sqrt(psq), pf_ref[...])
    tf_ref[...] = tf
    pf_ref[...] = pf
    e = jnp.abs((tf - pf) / (tf + pf + jnp.float32(1e-5)))
    acc_ref[0] += jnp.sum(e)

    @pl.when(i == n - 1)
    def _fin():
        out_ref[0] = acc_ref[0]


def kernel(predictions, target, alpha):
    b, c, h, w = predictions.shape
    n = b * c
    hw = h * w
    ax = int(np.floor(0.1 * hw))
    px = (h // NSUB) * w

    imgs = jnp.concatenate(
        [target.reshape(n, h, w), predictions.reshape(n, h, w)], axis=0)
    sq, sqb = _sq_maps(imgs)
    thr_raw = _sc_thresholds(
        sqb, n_img=n, hwsz=hw, px=px, k_rank=ax)
    thrs = thr_raw[:, :n, 0].reshape(2 * n)

    total = pl.pallas_call(
        _merge_body,
        grid=(n,),
        in_specs=[
            pl.BlockSpec((1, h, w), lambda i: (i, 0, 0)),
            pl.BlockSpec((1, h, w), lambda i: (i + n, 0, 0)),
            pl.BlockSpec(memory_space=pltpu.SMEM),
        ],
        out_specs=pl.BlockSpec(memory_space=pltpu.SMEM),
        out_shape=jax.ShapeDtypeStruct((1,), jnp.float32),
        scratch_shapes=[
            pltpu.VMEM((h, w), jnp.float32),
            pltpu.VMEM((h, w), jnp.float32),
            pltpu.SMEM((1,), jnp.float32),
        ],
    )(sq, sq, thrs)
    return alpha * total[0] / jnp.float32(n * hw)
